# baseline (speedup 1.0000x reference)
import functools

import jax
import jax.numpy as jnp
from jax import lax
from jax.experimental import pallas as pl
from jax.experimental.pallas import tpu as pltpu


def _round_up(x, m):
    return ((x + m - 1) // m) * m


def _lstm_dir_kernel(x_ref, wih_ref, whh_ref, b_ref, e_ref, xp,
                     *, seq_len, batch, hidden):
    T, Bp, H = seq_len, batch, hidden
    d = pl.program_id(0)

    xp[...] = (jnp.dot(x_ref[...], wih_ref[0],
                       preferred_element_type=jnp.float32) + b_ref[0])
    whh = whh_ref[0]

    def step(t, carry):
        h, c = carry
        ta = jnp.where(d == 0, t, T - 1 - t)
        row = pl.multiple_of(ta * Bp, Bp)
        pre = xp[pl.ds(row, Bp), :] + jnp.dot(
            h.astype(jnp.bfloat16), whh, preferred_element_type=jnp.float32)
        i = jax.nn.sigmoid(pre[:, 0:H])
        f = jax.nn.sigmoid(pre[:, H:2 * H])
        g = jnp.tanh(pre[:, 2 * H:3 * H])
        o = jax.nn.sigmoid(pre[:, 3 * H:4 * H])
        c = f * c + i * g
        h = o * jnp.tanh(c)
        e_ref[pl.ds(row, Bp), :] = h
        return h, c

    z = jnp.zeros((Bp, H), jnp.float32)
    lax.fori_loop(0, T, step, (z, z), unroll=True)


def _head_kernel(e_ref, gum_ref, wb_ref, bb_ref, wd_ref, il_ref, lg_ref,
                 *, inv_temp, n_gumbel, gp):
    il = (jnp.dot(e_ref[...].astype(jnp.bfloat16), wb_ref[...],
                  preferred_element_type=jnp.float32) + bb_ref[...])
    il_ref[...] = il
    y = (il + gum_ref[...]) * inv_temp
    if n_gumbel < gp:
        lane = lax.broadcasted_iota(jnp.int32, y.shape, 1)
        y = jnp.where(lane < n_gumbel, y, jnp.float32(-1e30))
    m = jnp.max(y, axis=-1, keepdims=True)
    e = jnp.exp(y - m)
    s = jnp.sum(e, axis=-1, keepdims=True)
    enc = e * pl.reciprocal(s, approx=True)
    lg_ref[...] = jnp.dot(enc.astype(jnp.bfloat16), wd_ref[...],
                          preferred_element_type=jnp.float32)


def kernel(x, wih_f, whh_f, b_f, wih_b, whh_b, b_b, wb, bias_b, wd,
           gumbel_noise):
    B, F, T = x.shape
    H = whh_f.shape[0]
    G = wb.shape[-1]
    C = wd.shape[-1]
    Bp = _round_up(max(B, 8), 8)
    Gp = _round_up(max(G, 128), 128)
    Cp = _round_up(max(C, 128), 128)
    TBp = T * Bp

    x_tbf = jnp.transpose(x, (2, 0, 1))
    x_tbf = jnp.pad(x_tbf, ((0, 0), (0, Bp - B), (0, 0)))
    x_2d = x_tbf.reshape(TBp, F).astype(jnp.bfloat16)

    wih = jnp.stack([wih_f, wih_b]).astype(jnp.bfloat16)
    whh = jnp.stack([whh_f, whh_b]).astype(jnp.bfloat16)
    bias = jnp.stack([b_f, b_b])

    lstm = functools.partial(_lstm_dir_kernel, seq_len=T, batch=Bp, hidden=H)
    embed = pl.pallas_call(
        lstm,
        grid=(2,),
        out_shape=jax.ShapeDtypeStruct((TBp, 2 * H), jnp.float32),
        in_specs=[
            pl.BlockSpec((TBp, F), lambda i: (0, 0)),
            pl.BlockSpec((1, F, 4 * H), lambda i: (i, 0, 0)),
            pl.BlockSpec((1, H, 4 * H), lambda i: (i, 0, 0)),
            pl.BlockSpec((1, 1, 4 * H), lambda i: (i, 0, 0)),
        ],
        out_specs=pl.BlockSpec((TBp, H), lambda i: (0, i)),
        scratch_shapes=[pltpu.VMEM((TBp, 4 * H), jnp.float32)],
        compiler_params=pltpu.CompilerParams(
            dimension_semantics=("parallel",)),
    )(x_2d, wih, whh, bias)

    gum = jnp.transpose(gumbel_noise, (1, 0, 2))
    gum = jnp.pad(gum, ((0, 0), (0, Bp - B), (0, Gp - G)))
    gum_2d = gum.reshape(TBp, Gp)

    wb_p = jnp.pad(wb, ((0, 0), (0, Gp - G))).astype(jnp.bfloat16)
    bb_p = jnp.pad(bias_b, ((0, 0), (0, Gp - G)))
    wd_p = jnp.pad(wd, ((0, Gp - G), (0, Cp - C))).astype(jnp.bfloat16)

    R = TBp // 2
    head = functools.partial(_head_kernel, inv_temp=1.0, n_gumbel=G, gp=Gp)
    il2, lg2 = pl.pallas_call(
        head,
        grid=(2,),
        out_shape=(jax.ShapeDtypeStruct((TBp, Gp), jnp.float32),
                   jax.ShapeDtypeStruct((TBp, Cp), jnp.float32)),
        in_specs=[
            pl.BlockSpec((R, 2 * H), lambda i: (i, 0)),
            pl.BlockSpec((R, Gp), lambda i: (i, 0)),
            pl.BlockSpec((2 * H, Gp), lambda i: (0, 0)),
            pl.BlockSpec((1, Gp), lambda i: (0, 0)),
            pl.BlockSpec((Gp, Cp), lambda i: (0, 0)),
        ],
        out_specs=(pl.BlockSpec((R, Gp), lambda i: (i, 0)),
                   pl.BlockSpec((R, Cp), lambda i: (i, 0))),
        compiler_params=pltpu.CompilerParams(
            dimension_semantics=("parallel",)),
    )(embed, gum_2d, wb_p, bb_p, wd_p)

    in_logit = jnp.transpose(il2.reshape(T, Bp, Gp)[:, :B, :G], (1, 0, 2))
    logit = jnp.transpose(lg2.reshape(T, Bp, Cp)[:, :B, :C], (1, 0, 2))
    return in_logit, logit


# speedup vs baseline: 1.4088x; 1.4088x over previous
import functools

import jax
import jax.numpy as jnp
from jax import lax
from jax.experimental import pallas as pl
from jax.experimental.pallas import tpu as pltpu


def _round_up(x, m):
    return ((x + m - 1) // m) * m


def _lstm_dir_kernel(x_ref, wih_ref, whh_ref, b_ref, e_ref, xp,
                     *, seq_len, batch, hidden):
    T, Bp, H = seq_len, batch, hidden
    d = pl.program_id(0)

    xp[...] = (jnp.dot(x_ref[...], wih_ref[0],
                       preferred_element_type=jnp.float32) + b_ref[0])
    whh = whh_ref[0]

    def step(t, carry):
        h, c = carry
        ta = jnp.where(d == 0, t, T - 1 - t)
        row = pl.multiple_of(ta * Bp, Bp)
        pre = xp[pl.ds(row, Bp), :] + jnp.dot(
            h.astype(jnp.bfloat16), whh, preferred_element_type=jnp.float32)
        i = jax.nn.sigmoid(pre[:, 0:H])
        f = jax.nn.sigmoid(pre[:, H:2 * H])
        g = jnp.tanh(pre[:, 2 * H:3 * H])
        o = jax.nn.sigmoid(pre[:, 3 * H:4 * H])
        c = f * c + i * g
        h = o * jnp.tanh(c)
        e_ref[pl.ds(row, Bp), :] = h.astype(jnp.bfloat16)
        return h, c

    z = jnp.zeros((Bp, H), jnp.float32)
    lax.fori_loop(0, T, step, (z, z), unroll=True)


def _head_kernel(e_ref, gum_ref, wb_ref, bb_ref, wd_ref, il_ref, lg_ref,
                 *, inv_temp, n_gumbel, gp):
    il = (jnp.dot(e_ref[...], wb_ref[...],
                  preferred_element_type=jnp.float32) + bb_ref[...])
    il_ref[...] = il
    y = (il + gum_ref[...]) * inv_temp
    if n_gumbel < gp:
        lane = lax.broadcasted_iota(jnp.int32, y.shape, 1)
        y = jnp.where(lane < n_gumbel, y, jnp.float32(-1e30))
    m = jnp.max(y, axis=-1, keepdims=True)
    e = jnp.exp(y - m)
    s = jnp.sum(e, axis=-1, keepdims=True)
    enc = e * pl.reciprocal(s, approx=True)
    lg_ref[...] = jnp.dot(enc.astype(jnp.bfloat16), wd_ref[...],
                          preferred_element_type=jnp.float32)


def kernel(x, wih_f, whh_f, b_f, wih_b, whh_b, b_b, wb, bias_b, wd,
           gumbel_noise):
    B, F, T = x.shape
    H = whh_f.shape[0]
    G = wb.shape[-1]
    C = wd.shape[-1]
    Bp = _round_up(max(B, 8), 8)
    Gp = _round_up(max(G, 128), 128)
    Cp = _round_up(max(C, 128), 128)
    TBp = T * Bp

    x_tbf = jnp.transpose(x.astype(jnp.bfloat16), (2, 0, 1))
    x_tbf = jnp.pad(x_tbf, ((0, 0), (0, Bp - B), (0, 0)))
    x_2d = x_tbf.reshape(TBp, F)

    wih = jnp.stack([wih_f, wih_b]).astype(jnp.bfloat16)
    whh = jnp.stack([whh_f, whh_b]).astype(jnp.bfloat16)
    bias = jnp.stack([b_f, b_b])

    lstm = functools.partial(_lstm_dir_kernel, seq_len=T, batch=Bp, hidden=H)
    embed_tm = pl.pallas_call(
        lstm,
        grid=(2,),
        out_shape=jax.ShapeDtypeStruct((TBp, 2 * H), jnp.bfloat16),
        in_specs=[
            pl.BlockSpec((TBp, F), lambda i: (0, 0)),
            pl.BlockSpec((1, F, 4 * H), lambda i: (i, 0, 0)),
            pl.BlockSpec((1, H, 4 * H), lambda i: (i, 0, 0)),
            pl.BlockSpec((1, 1, 4 * H), lambda i: (i, 0, 0)),
        ],
        out_specs=pl.BlockSpec((TBp, H), lambda i: (0, i)),
        scratch_shapes=[pltpu.VMEM((TBp, 4 * H), jnp.float32)],
        compiler_params=pltpu.CompilerParams(
            dimension_semantics=("parallel",)),
    )(x_2d, wih, whh, bias)

    e_bm = jnp.transpose(embed_tm.reshape(T, Bp, 2 * H),
                         (1, 0, 2)).reshape(Bp * T, 2 * H)

    gum_2d = gumbel_noise.reshape(B * T, G)
    gum_2d = jnp.pad(gum_2d, ((0, (Bp - B) * T), (0, Gp - G)))

    wb_p = jnp.pad(wb, ((0, 0), (0, Gp - G))).astype(jnp.bfloat16)
    bb_p = jnp.pad(bias_b, ((0, 0), (0, Gp - G)))
    wd_p = jnp.pad(wd, ((0, Gp - G), (0, Cp - C))).astype(jnp.bfloat16)

    R = TBp // 2
    head = functools.partial(_head_kernel, inv_temp=1.0, n_gumbel=G, gp=Gp)
    il2, lg2 = pl.pallas_call(
        head,
        grid=(2,),
        out_shape=(jax.ShapeDtypeStruct((TBp, Gp), jnp.float32),
                   jax.ShapeDtypeStruct((TBp, Cp), jnp.float32)),
        in_specs=[
            pl.BlockSpec((R, 2 * H), lambda i: (i, 0)),
            pl.BlockSpec((R, Gp), lambda i: (i, 0)),
            pl.BlockSpec((2 * H, Gp), lambda i: (0, 0)),
            pl.BlockSpec((1, Gp), lambda i: (0, 0)),
            pl.BlockSpec((Gp, Cp), lambda i: (0, 0)),
        ],
        out_specs=(pl.BlockSpec((R, Gp), lambda i: (i, 0)),
                   pl.BlockSpec((R, Cp), lambda i: (i, 0))),
        compiler_params=pltpu.CompilerParams(
            dimension_semantics=("parallel",)),
    )(e_bm, gum_2d, wb_p, bb_p, wd_p)

    in_logit = il2.reshape(Bp, T, Gp)[:B, :, :G]
    logit = lg2.reshape(Bp, T, Cp)[:B, :, :C]
    return in_logit, logit


# speedup vs baseline: 1.5914x; 1.1296x over previous
import functools

import jax
import jax.numpy as jnp
from jax import lax
from jax.experimental import pallas as pl
from jax.experimental.pallas import tpu as pltpu


def _round_up(x, m):
    return ((x + m - 1) // m) * m


def _lstm_dir_kernel(x_ref, wih_f_ref, whh_f_ref, b_f_ref,
                     wih_b_ref, whh_b_ref, b_b_ref, e_ref, xp,
                     *, seq_len, batch, hidden):
    T, Bp, H = seq_len, batch, hidden
    d = pl.program_id(0)
    fwd = d == 0

    wih = jnp.where(fwd, wih_f_ref[...], wih_b_ref[...]).astype(jnp.bfloat16)
    whh = jnp.where(fwd, whh_f_ref[...], whh_b_ref[...]).astype(jnp.bfloat16)
    b = jnp.where(fwd, b_f_ref[...], b_b_ref[...])

    xp[...] = (jnp.dot(x_ref[...], wih,
                       preferred_element_type=jnp.float32) + b)

    def step(t, carry):
        h, c = carry
        ta = jnp.where(d == 0, t, T - 1 - t)
        row = pl.multiple_of(ta * Bp, Bp)
        pre = xp[pl.ds(row, Bp), :] + jnp.dot(
            h.astype(jnp.bfloat16), whh, preferred_element_type=jnp.float32)
        i = jax.nn.sigmoid(pre[:, 0:H])
        f = jax.nn.sigmoid(pre[:, H:2 * H])
        g = jnp.tanh(pre[:, 2 * H:3 * H])
        o = jax.nn.sigmoid(pre[:, 3 * H:4 * H])
        c = f * c + i * g
        h = o * jnp.tanh(c)
        e_ref[pl.ds(row, Bp), :] = h.astype(jnp.bfloat16)
        return h, c

    z = jnp.zeros((Bp, H), jnp.float32)
    lax.fori_loop(0, T, step, (z, z), unroll=True)


def _head_kernel(e_ref, gum_ref, wb_ref, bb_ref, wd_ref, il_ref, lg_ref,
                 *, inv_temp, n_gumbel, gp):
    il = (jnp.dot(e_ref[...], wb_ref[...].astype(jnp.bfloat16),
                  preferred_element_type=jnp.float32) + bb_ref[...])
    il_ref[...] = il
    y = (il + gum_ref[...]) * inv_temp
    if n_gumbel < gp:
        lane = lax.broadcasted_iota(jnp.int32, y.shape, 1)
        y = jnp.where(lane < n_gumbel, y, jnp.float32(-1e30))
    m = jnp.max(y, axis=-1, keepdims=True)
    e = jnp.exp(y - m)
    s = jnp.sum(e, axis=-1, keepdims=True)
    enc = e * pl.reciprocal(s, approx=True)
    lg_ref[...] = jnp.dot(enc.astype(jnp.bfloat16),
                          wd_ref[...].astype(jnp.bfloat16),
                          preferred_element_type=jnp.float32)


def kernel(x, wih_f, whh_f, b_f, wih_b, whh_b, b_b, wb, bias_b, wd,
           gumbel_noise):
    B, F, T = x.shape
    H = whh_f.shape[0]
    G = wb.shape[-1]
    C = wd.shape[-1]
    Bp = _round_up(max(B, 8), 8)
    Gp = _round_up(max(G, 128), 128)
    Cp = _round_up(max(C, 128), 128)
    TBp = T * Bp

    x_tbf = jnp.transpose(x.astype(jnp.bfloat16), (2, 0, 1))
    x_tbf = jnp.pad(x_tbf, ((0, 0), (0, Bp - B), (0, 0)))
    x_2d = x_tbf.reshape(TBp, F)

    lstm = functools.partial(_lstm_dir_kernel, seq_len=T, batch=Bp, hidden=H)
    full2 = lambda i: (0, 0)
    embed_tm = pl.pallas_call(
        lstm,
        grid=(2,),
        out_shape=jax.ShapeDtypeStruct((TBp, 2 * H), jnp.bfloat16),
        in_specs=[
            pl.BlockSpec((TBp, F), full2),
            pl.BlockSpec((F, 4 * H), full2),
            pl.BlockSpec((H, 4 * H), full2),
            pl.BlockSpec((1, 4 * H), full2),
            pl.BlockSpec((F, 4 * H), full2),
            pl.BlockSpec((H, 4 * H), full2),
            pl.BlockSpec((1, 4 * H), full2),
        ],
        out_specs=pl.BlockSpec((TBp, H), lambda i: (0, i)),
        scratch_shapes=[pltpu.VMEM((TBp, 4 * H), jnp.float32)],
        compiler_params=pltpu.CompilerParams(
            dimension_semantics=("parallel",)),
    )(x_2d, wih_f, whh_f, b_f, wih_b, whh_b, b_b)

    e_bm = jnp.transpose(embed_tm.reshape(T, Bp, 2 * H),
                         (1, 0, 2)).reshape(Bp * T, 2 * H)

    gum_2d = gumbel_noise.reshape(B * T, G)
    gum_2d = jnp.pad(gum_2d, ((0, (Bp - B) * T), (0, Gp - G)))

    wb_p = jnp.pad(wb, ((0, 0), (0, Gp - G)))
    bb_p = jnp.pad(bias_b, ((0, 0), (0, Gp - G)))
    wd_p = jnp.pad(wd, ((0, Gp - G), (0, Cp - C)))

    R = TBp // 2
    head = functools.partial(_head_kernel, inv_temp=1.0, n_gumbel=G, gp=Gp)
    il2, lg2 = pl.pallas_call(
        head,
        grid=(2,),
        out_shape=(jax.ShapeDtypeStruct((TBp, Gp), jnp.float32),
                   jax.ShapeDtypeStruct((TBp, Cp), jnp.float32)),
        in_specs=[
            pl.BlockSpec((R, 2 * H), lambda i: (i, 0)),
            pl.BlockSpec((R, Gp), lambda i: (i, 0)),
            pl.BlockSpec((2 * H, Gp), lambda i: (0, 0)),
            pl.BlockSpec((1, Gp), lambda i: (0, 0)),
            pl.BlockSpec((Gp, Cp), lambda i: (0, 0)),
        ],
        out_specs=(pl.BlockSpec((R, Gp), lambda i: (i, 0)),
                   pl.BlockSpec((R, Cp), lambda i: (i, 0))),
        compiler_params=pltpu.CompilerParams(
            dimension_semantics=("parallel",)),
    )(e_bm, gum_2d, wb_p, bb_p, wd_p)

    in_logit = il2.reshape(Bp, T, Gp)[:B, :, :G]
    logit = lg2.reshape(Bp, T, Cp)[:B, :, :C]
    return in_logit, logit
